# Initial kernel scaffold; baseline (speedup 1.0000x reference)
import jax
import jax.numpy as jnp
from jax.experimental import pallas as pl


def kernel(x, a, W1, b1, W2, b2, W3, b3, W4, b4, D1W, D1b, D2W, D2b, DtW, Dtb, ZmW, Zmb, ZvW, Zvb):
    raise NotImplementedError("write your pallas kernel here")


# speedup vs baseline: 1.2954x; 1.2954x over previous
import jax
import jax.numpy as jnp
from jax.experimental import pallas as pl

B, N, F = 256, 128, 64
H = 256
H2 = 128
L = 64

BB = 8


def _dot(a_, b_):
    return jnp.dot(a_, b_, preferred_element_type=jnp.float32)


def _conv_kernel(x_ref, a_ref, w1_ref, b1_ref, w2_ref, b2_ref,
                 w3_ref, b3_ref, w4_ref, b4_ref, out_ref):
    h = x_ref[...].reshape(BB * N, F)

    def gcn(h2d, w_ref, b_ref):
        m = _dot(h2d, w_ref[...])
        b = b_ref[...]
        outs = []
        for g in range(BB):
            ag = a_ref[g]
            mg = m[g * N:(g + 1) * N]
            outs.append(jax.nn.relu(_dot(ag, mg) + b))
        return jnp.concatenate(outs, axis=0)

    h = gcn(h, w1_ref, b1_ref)
    h = gcn(h, w2_ref, b2_ref)
    h = gcn(h, w3_ref, b3_ref)
    h = gcn(h, w4_ref, b4_ref)
    out_ref[...] = h.reshape(BB, N, H2)


def _dense_kernel(h_ref, d1w_ref, d1b_ref, d2w_ref, d2b_ref,
                  dtw_ref, dtb_ref, zmw_ref, zmb_ref, zvw_ref, zvb_ref,
                  eps_ref, zm_ref, zv_ref, z_ref):
    h = jax.nn.relu(_dot(h_ref[...], d1w_ref[...]) + d1b_ref[...])
    h = jax.nn.relu(_dot(h, d2w_ref[...]) + d2b_ref[...])
    t = jnp.tanh(_dot(h, dtw_ref[...]) + dtb_ref[...])
    zm = jax.nn.relu(_dot(t, zmw_ref[...]) + zmb_ref[...])
    zv = jax.nn.relu(_dot(t, zvw_ref[...]) + zvb_ref[...])
    zm_ref[...] = zm
    zv_ref[...] = zv
    z_ref[...] = zm + jnp.exp(0.5 * zv) * eps_ref[...]


def kernel(x, a, W1, b1, W2, b2, W3, b3, W4, b4,
           D1W, D1b, D2W, D2b, DtW, Dtb, ZmW, Zmb, ZvW, Zvb):
    f32 = jnp.float32
    row = lambda v: v.reshape(1, -1)

    full = lambda s: pl.BlockSpec(s, lambda i: (0,) * len(s))
    batched = lambda s: pl.BlockSpec(s, lambda i: (i,) + (0,) * (len(s) - 1))

    h4 = pl.pallas_call(
        _conv_kernel,
        grid=(B // BB,),
        in_specs=[
            batched((BB, N, F)),
            batched((BB, N, N)),
            full((F, H)), full((1, H)),
            full((H, H)), full((1, H)),
            full((H, H)), full((1, H)),
            full((H, H2)), full((1, H2)),
        ],
        out_specs=batched((BB, N, H2)),
        out_shape=jax.ShapeDtypeStruct((B, N, H2), f32),
    )(x, a, W1, row(b1), W2, row(b2), W3, row(b3), W4, row(b4))

    h4f = h4.reshape(B, N * H2)
    eps = jax.random.normal(jax.random.key(1), (B, L), dtype=f32)

    out_shapes = [jax.ShapeDtypeStruct((B, L), f32)] * 3
    zm, zv, z = pl.pallas_call(
        _dense_kernel,
        out_shape=out_shapes,
    )(h4f, D1W, row(D1b), D2W, row(D2b), DtW, row(Dtb),
      ZmW, row(Zmb), ZvW, row(Zvb), eps)
    return (zm, zv, z)


# speedup vs baseline: 1.5722x; 1.2137x over previous
import jax
import jax.numpy as jnp
from jax.experimental import pallas as pl

B, N, F = 256, 128, 64
H = 256
H2 = 128
L = 64

BB = 8


def _dot(a_, b_):
    return jnp.dot(a_, b_, preferred_element_type=jnp.float32)


def _conv_kernel(x_ref, a_ref, w1_ref, b1_ref, w2_ref, b2_ref,
                 w3_ref, b3_ref, w4_ref, b4_ref, out_ref):
    h = x_ref[...].reshape(BB * N, F)

    def gcn(h2d, w_ref, b_ref):
        m = _dot(h2d, w_ref[...])
        b = b_ref[...]
        outs = []
        for g in range(BB):
            ag = a_ref[g]
            mg = m[g * N:(g + 1) * N]
            outs.append(jax.nn.relu(_dot(ag, mg) + b))
        return jnp.concatenate(outs, axis=0)

    h = gcn(h, w1_ref, b1_ref)
    h = gcn(h, w2_ref, b2_ref)
    h = gcn(h, w3_ref, b3_ref)
    h = gcn(h, w4_ref, b4_ref)
    out_ref[...] = h.reshape(BB, N, H2)


def _dense_kernel(h_ref, d1w_ref, d1b_ref, d2w_ref, d2b_ref,
                  dtw_ref, dtb_ref, zmw_ref, zmb_ref, zvw_ref, zvb_ref,
                  eps_ref, zm_ref, zv_ref, z_ref):
    acc = d1b_ref[...] * jnp.ones((B, 1), jnp.float32)
    for n in range(N):
        acc = acc + _dot(h_ref[:, n, :], d1w_ref[n * H2:(n + 1) * H2, :])
    h = jax.nn.relu(acc)
    h = jax.nn.relu(_dot(h, d2w_ref[...]) + d2b_ref[...])
    t = jnp.tanh(_dot(h, dtw_ref[...]) + dtb_ref[...])
    zm = jax.nn.relu(_dot(t, zmw_ref[...]) + zmb_ref[...])
    zv = jax.nn.relu(_dot(t, zvw_ref[...]) + zvb_ref[...])
    zm_ref[...] = zm
    zv_ref[...] = zv
    z_ref[...] = zm + jnp.exp(0.5 * zv) * eps_ref[...]


def kernel(x, a, W1, b1, W2, b2, W3, b3, W4, b4,
           D1W, D1b, D2W, D2b, DtW, Dtb, ZmW, Zmb, ZvW, Zvb):
    f32 = jnp.float32
    row = lambda v: v.reshape(1, -1)

    full = lambda s: pl.BlockSpec(s, lambda i: (0,) * len(s))
    batched = lambda s: pl.BlockSpec(s, lambda i: (i,) + (0,) * (len(s) - 1))

    h4 = pl.pallas_call(
        _conv_kernel,
        grid=(B // BB,),
        in_specs=[
            batched((BB, N, F)),
            batched((BB, N, N)),
            full((F, H)), full((1, H)),
            full((H, H)), full((1, H)),
            full((H, H)), full((1, H)),
            full((H, H2)), full((1, H2)),
        ],
        out_specs=batched((BB, N, H2)),
        out_shape=jax.ShapeDtypeStruct((B, N, H2), f32),
    )(x, a, W1, row(b1), W2, row(b2), W3, row(b3), W4, row(b4))

    eps = jax.random.normal(jax.random.key(1), (B, L), dtype=f32)

    out_shapes = [jax.ShapeDtypeStruct((B, L), f32)] * 3
    zm, zv, z = pl.pallas_call(
        _dense_kernel,
        out_shape=out_shapes,
    )(h4, D1W, row(D1b), D2W, row(D2b), DtW, row(Dtb),
      ZmW, row(Zmb), ZvW, row(Zvb), eps)
    return (zm, zv, z)


# speedup vs baseline: 1.7167x; 1.0919x over previous
import jax
import jax.numpy as jnp
from jax.experimental import pallas as pl

B, N, F = 256, 128, 64
H = 256
H2 = 128
L = 64

BB = 16


def _dot(a_, b_):
    return jnp.dot(a_, b_, preferred_element_type=jnp.float32)


def _conv_kernel(x_ref, a_ref, w1_ref, b1_ref, w2_ref, b2_ref,
                 w3_ref, b3_ref, w4_ref, b4_ref, out_ref):
    h = x_ref[...].reshape(BB * N, F)

    def gcn(h2d, w_ref, b_ref):
        m = _dot(h2d, w_ref[...])
        b = b_ref[...]
        outs = []
        for g in range(BB):
            ag = a_ref[g]
            mg = m[g * N:(g + 1) * N]
            outs.append(jax.nn.relu(_dot(ag, mg) + b))
        return jnp.concatenate(outs, axis=0)

    h = gcn(h, w1_ref, b1_ref)
    h = gcn(h, w2_ref, b2_ref)
    h = gcn(h, w3_ref, b3_ref)
    h = gcn(h, w4_ref, b4_ref)
    out_ref[...] = h.reshape(BB, N, H2)


def _dense_kernel(h_ref, d1w_ref, d1b_ref, d2w_ref, d2b_ref,
                  dtw_ref, dtb_ref, zmw_ref, zmb_ref, zvw_ref, zvb_ref,
                  eps_ref, zm_ref, zv_ref, z_ref):
    acc = d1b_ref[...] * jnp.ones((B, 1), jnp.float32)
    for n in range(N):
        acc = acc + _dot(h_ref[:, n, :], d1w_ref[n * H2:(n + 1) * H2, :])
    h = jax.nn.relu(acc)
    h = jax.nn.relu(_dot(h, d2w_ref[...]) + d2b_ref[...])
    t = jnp.tanh(_dot(h, dtw_ref[...]) + dtb_ref[...])
    zm = jax.nn.relu(_dot(t, zmw_ref[...]) + zmb_ref[...])
    zv = jax.nn.relu(_dot(t, zvw_ref[...]) + zvb_ref[...])
    zm_ref[...] = zm
    zv_ref[...] = zv
    z_ref[...] = zm + jnp.exp(0.5 * zv) * eps_ref[...]


def kernel(x, a, W1, b1, W2, b2, W3, b3, W4, b4,
           D1W, D1b, D2W, D2b, DtW, Dtb, ZmW, Zmb, ZvW, Zvb):
    f32 = jnp.float32
    row = lambda v: v.reshape(1, -1)

    full = lambda s: pl.BlockSpec(s, lambda i: (0,) * len(s))
    batched = lambda s: pl.BlockSpec(s, lambda i: (i,) + (0,) * (len(s) - 1))

    h4 = pl.pallas_call(
        _conv_kernel,
        grid=(B // BB,),
        in_specs=[
            batched((BB, N, F)),
            batched((BB, N, N)),
            full((F, H)), full((1, H)),
            full((H, H)), full((1, H)),
            full((H, H)), full((1, H)),
            full((H, H2)), full((1, H2)),
        ],
        out_specs=batched((BB, N, H2)),
        out_shape=jax.ShapeDtypeStruct((B, N, H2), f32),
    )(x, a, W1, row(b1), W2, row(b2), W3, row(b3), W4, row(b4))

    eps = jax.random.normal(jax.random.key(1), (B, L), dtype=f32)

    out_shapes = [jax.ShapeDtypeStruct((B, L), f32)] * 3
    zm, zv, z = pl.pallas_call(
        _dense_kernel,
        out_shape=out_shapes,
    )(h4, D1W, row(D1b), D2W, row(D2b), DtW, row(Dtb),
      ZmW, row(Zmb), ZvW, row(Zvb), eps)
    return (zm, zv, z)


# speedup vs baseline: 1.7608x; 1.0257x over previous
import jax
import jax.numpy as jnp
from jax.experimental import pallas as pl

B, N, F = 256, 128, 64
H = 256
H2 = 128
L = 64

BB = 32


def _dot(a_, b_):
    return jnp.dot(a_, b_, preferred_element_type=jnp.float32)


def _conv_kernel(x_ref, a_ref, w1_ref, b1_ref, w2_ref, b2_ref,
                 w3_ref, b3_ref, w4_ref, b4_ref, out_ref):
    h = x_ref[...].reshape(BB * N, F)

    def gcn(h2d, w_ref, b_ref):
        m = _dot(h2d, w_ref[...])
        b = b_ref[...]
        outs = []
        for g in range(BB):
            ag = a_ref[g]
            mg = m[g * N:(g + 1) * N]
            outs.append(jax.nn.relu(_dot(ag, mg) + b))
        return jnp.concatenate(outs, axis=0)

    h = gcn(h, w1_ref, b1_ref)
    h = gcn(h, w2_ref, b2_ref)
    h = gcn(h, w3_ref, b3_ref)
    h = gcn(h, w4_ref, b4_ref)
    out_ref[...] = h.reshape(BB, N, H2)


def _dense_kernel(h_ref, d1w_ref, d1b_ref, d2w_ref, d2b_ref,
                  dtw_ref, dtb_ref, zmw_ref, zmb_ref, zvw_ref, zvb_ref,
                  eps_ref, zm_ref, zv_ref, z_ref):
    acc = d1b_ref[...] * jnp.ones((B, 1), jnp.float32)
    for n in range(N):
        acc = acc + _dot(h_ref[:, n, :], d1w_ref[n * H2:(n + 1) * H2, :])
    h = jax.nn.relu(acc)
    h = jax.nn.relu(_dot(h, d2w_ref[...]) + d2b_ref[...])
    t = jnp.tanh(_dot(h, dtw_ref[...]) + dtb_ref[...])
    zm = jax.nn.relu(_dot(t, zmw_ref[...]) + zmb_ref[...])
    zv = jax.nn.relu(_dot(t, zvw_ref[...]) + zvb_ref[...])
    zm_ref[...] = zm
    zv_ref[...] = zv
    z_ref[...] = zm + jnp.exp(0.5 * zv) * eps_ref[...]


def kernel(x, a, W1, b1, W2, b2, W3, b3, W4, b4,
           D1W, D1b, D2W, D2b, DtW, Dtb, ZmW, Zmb, ZvW, Zvb):
    f32 = jnp.float32
    row = lambda v: v.reshape(1, -1)

    full = lambda s: pl.BlockSpec(s, lambda i: (0,) * len(s))
    batched = lambda s: pl.BlockSpec(s, lambda i: (i,) + (0,) * (len(s) - 1))

    h4 = pl.pallas_call(
        _conv_kernel,
        grid=(B // BB,),
        in_specs=[
            batched((BB, N, F)),
            batched((BB, N, N)),
            full((F, H)), full((1, H)),
            full((H, H)), full((1, H)),
            full((H, H)), full((1, H)),
            full((H, H2)), full((1, H2)),
        ],
        out_specs=batched((BB, N, H2)),
        out_shape=jax.ShapeDtypeStruct((B, N, H2), f32),
    )(x, a, W1, row(b1), W2, row(b2), W3, row(b3), W4, row(b4))

    eps = jax.random.normal(jax.random.key(1), (B, L), dtype=f32)

    out_shapes = [jax.ShapeDtypeStruct((B, L), f32)] * 3
    zm, zv, z = pl.pallas_call(
        _dense_kernel,
        out_shape=out_shapes,
    )(h4, D1W, row(D1b), D2W, row(D2b), DtW, row(Dtb),
      ZmW, row(Zmb), ZvW, row(Zvb), eps)
    return (zm, zv, z)
